# baseline (speedup 1.0000x reference)
import jax
import jax.numpy as jnp
from jax import lax
from jax.experimental import pallas as pl
from jax.experimental.pallas import tpu as pltpu
from jax.experimental.pallas import tpu_sc as plsc

_B, _S = 16, 2048
_ON, _OL = 32, 64
_DELIM, _PAD = 1, 0
_L = 16
_ROW_LEN = _S + _L


def _split_body(x_hbm, otp_hbm, lend_hbm, mask_hbm,
                row_v, beg_v, out_v, msk_v, sem_out, sem_msk):
    cid = lax.axis_index("c")
    sid = lax.axis_index("s")
    row = cid * 8 + sid

    @pl.when(sid < 8)
    def _():
        lane = lax.broadcasted_iota(jnp.int32, (_L,), 0)
        pieces = [pltpu.async_copy(x_hbm.at[row], row_v.at[pl.ds(0, _S)],
                                   sem_out)]
        row_v[pl.ds(_S, _L)] = jnp.where(lane == 0, _DELIM, _PAD)
        beg_v[pl.ds(0, _L)] = jnp.where(lane == 0, 0, _S + 1)
        beg_v[pl.ds(_L, _L)] = jnp.full((_L,), _S + 1, jnp.int32)
        beg_v[pl.ds(2 * _L, _L)] = jnp.full((_L,), _S + 1, jnp.int32)
        for c in pieces:
            c.wait()

        def scan_cond(c):
            i, found = c
            return (i < _S // _L) & (found < _ON - 1)

        def scan_body(c):
            i, found = c
            v = row_v[pl.ds(i * _L, _L)]
            m = v == _DELIM
            pc = plsc.all_reduce_population_count(m)[0]

            @pl.when(pc != 0)
            def _():
                s = plsc.cumsum(jnp.where(m, 1, 0))
                rank = found + s
                wm = m & (rank <= _ON - 1)
                plsc.store_scatter(beg_v, [jnp.minimum(rank, _ON)],
                                   i * _L + lane + 1, mask=wm)

            return i + 1, found + pc

        lax.while_loop(scan_cond, scan_body, (jnp.int32(0), jnp.int32(0)))

        def chunk_body(k, doc):
            ksplat = jnp.zeros((_L,), jnp.int32) + k
            off = plsc.load_gather(beg_v, [ksplat])
            sz = plsc.load_gather(beg_v, [ksplat + 1]) - off
            ok = sz != 1
            ln = jnp.zeros((_L,), jnp.int32)
            for q in range(_OL // _L):
                j = lane + q * _L
                idx = jnp.minimum(off + j, _S)
                g = plsc.load_gather(row_v, [idx])
                val = jnp.where((j < sz) & ok, g, _PAD)
                out_v[k, pl.ds(q * _L, _L)] = val
                ln = ln + plsc.all_reduce_population_count(val != 0)
            for q in range(_OL // _L):
                j = lane + q * _L
                msk_v[k, pl.ds(q * _L, _L)] = jnp.where(
                    j < ln, jnp.float32(1.0), jnp.float32(0.0))
            return doc + jnp.where(ln != 0, 1, 0)

        doc = lax.fori_loop(0, _ON, chunk_body, jnp.zeros((_L,), jnp.int32))
        c1 = pltpu.async_copy(out_v, otp_hbm.at[row], sem_out)
        c2 = pltpu.async_copy(msk_v, mask_hbm.at[row], sem_msk)
        beg_v[pl.ds(0, _L)] = doc
        pltpu.sync_copy(beg_v.at[pl.ds(0, 8)], lend_hbm.at[pl.ds(row * 8, 8)])
        c1.wait()
        c2.wait()


@jax.jit
def kernel(x):
    mesh = plsc.VectorSubcoreMesh(core_axis_name="c", subcore_axis_name="s")
    otp, lend, mask = pl.kernel(
        _split_body,
        out_type=[
            jax.ShapeDtypeStruct((_B, _ON, _OL), jnp.int32),
            jax.ShapeDtypeStruct((_B * 8,), jnp.int32),
            jax.ShapeDtypeStruct((_B, _ON, _OL), jnp.float32),
        ],
        mesh=mesh,
        compiler_params=pltpu.CompilerParams(
            needs_layout_passes=False, use_tc_tiling_on_sc=True),
        scratch_types=[
            pltpu.VMEM((_ROW_LEN,), jnp.int32),
            pltpu.VMEM((3 * _L,), jnp.int32),
            pltpu.VMEM((_ON, _OL), jnp.int32),
            pltpu.VMEM((_ON, _OL), jnp.float32),
            pltpu.SemaphoreType.DMA,
            pltpu.SemaphoreType.DMA,
        ],
    )(x)
    return otp, lend[::8], mask


# speedup vs baseline: 1.0614x; 1.0614x over previous
import jax
import jax.numpy as jnp
from jax import lax
from jax.experimental import pallas as pl
from jax.experimental.pallas import tpu as pltpu
from jax.experimental.pallas import tpu_sc as plsc

_B, _S = 16, 2048
_ON, _OL = 32, 64
_DELIM, _PAD = 1, 0
_L = 16
_ROW_LEN = _S + _L


def _split_body(x_hbm, otp_hbm, lend_hbm, mask_hbm,
                row_v, beg_v, out_v, msk_v, docs_sh, tmp_v, sem_out, sem_msk):
    cid = lax.axis_index("c")
    sid = lax.axis_index("s")
    row = cid * 8 + sid // 2
    half = sid % 2

    if True:
        lane = lax.broadcasted_iota(jnp.int32, (_L,), 0)
        pieces = [pltpu.async_copy(x_hbm.at[row], row_v.at[pl.ds(0, _S)],
                                   sem_out)]
        row_v[pl.ds(_S, _L)] = jnp.where(lane == 0, _DELIM, _PAD)
        beg_v[pl.ds(0, _L)] = jnp.where(lane == 0, 0, _S + 1)
        beg_v[pl.ds(_L, _L)] = jnp.full((_L,), _S + 1, jnp.int32)
        beg_v[pl.ds(2 * _L, _L)] = jnp.full((_L,), _S + 1, jnp.int32)
        for c in pieces:
            c.wait()

        def scan_cond(c):
            i, found = c
            return (i < _S // _L) & (found < _ON - 1)

        def scan_body(c):
            i, found = c
            v = row_v[pl.ds(i * _L, _L)]
            m = v == _DELIM
            pc = plsc.all_reduce_population_count(m)[0]

            @pl.when(pc != 0)
            def _():
                s = plsc.cumsum(jnp.where(m, 1, 0))
                rank = found + s
                wm = m & (rank <= _ON - 1)
                plsc.store_scatter(beg_v, [jnp.minimum(rank, _ON)],
                                   i * _L + lane + 1, mask=wm)

            return i + 1, found + pc

        lax.while_loop(scan_cond, scan_body, (jnp.int32(0), jnp.int32(0)))

        kbase = half * (_ON // 2)

        def chunk_body(kk, doc):
            k = kbase + kk
            ksplat = jnp.zeros((_L,), jnp.int32) + k
            off = plsc.load_gather(beg_v, [ksplat])
            sz = plsc.load_gather(beg_v, [ksplat + 1]) - off
            ok = sz != 1
            ln = jnp.zeros((_L,), jnp.int32)
            for q in range(_OL // _L):
                j = lane + q * _L
                idx = jnp.minimum(off + j, _S)
                g = plsc.load_gather(row_v, [idx])
                val = jnp.where((j < sz) & ok, g, _PAD)
                out_v[kk, pl.ds(q * _L, _L)] = val
                ln = ln + plsc.all_reduce_population_count(val != 0)
            for q in range(_OL // _L):
                j = lane + q * _L
                msk_v[kk, pl.ds(q * _L, _L)] = jnp.where(
                    j < ln, jnp.float32(1.0), jnp.float32(0.0))
            return doc + jnp.where(ln != 0, 1, 0)

        doc = lax.fori_loop(0, _ON // 2, chunk_body,
                            jnp.zeros((_L,), jnp.int32))
        c1 = pltpu.async_copy(out_v, otp_hbm.at[row, pl.ds(kbase, _ON // 2)],
                              sem_out)
        c2 = pltpu.async_copy(msk_v, mask_hbm.at[row, pl.ds(kbase, _ON // 2)],
                              sem_msk)
        beg_v[pl.ds(0, _L)] = doc
        pltpu.sync_copy(beg_v.at[pl.ds(0, 8)], docs_sh.at[pl.ds(sid * 8, 8)])
        c1.wait()
        c2.wait()

    plsc.subcore_barrier()

    @pl.when(sid == 0)
    def _():
        pltpu.sync_copy(docs_sh, tmp_v)
        idx = lax.broadcasted_iota(jnp.int32, (_L,), 0) * 16
        cnt = (plsc.load_gather(tmp_v, [idx])
               + plsc.load_gather(tmp_v, [idx + 8]))
        tmp_v[pl.ds(0, _L)] = cnt
        pltpu.sync_copy(tmp_v.at[pl.ds(0, 8)], lend_hbm.at[pl.ds(cid * 8, 8)])


@jax.jit
def kernel(x):
    mesh = plsc.VectorSubcoreMesh(core_axis_name="c", subcore_axis_name="s")
    otp, lend, mask = pl.kernel(
        _split_body,
        out_type=[
            jax.ShapeDtypeStruct((_B, _ON, _OL), jnp.int32),
            jax.ShapeDtypeStruct((_B,), jnp.int32),
            jax.ShapeDtypeStruct((_B, _ON, _OL), jnp.float32),
        ],
        mesh=mesh,
        compiler_params=pltpu.CompilerParams(
            needs_layout_passes=False, use_tc_tiling_on_sc=True),
        scratch_types=[
            pltpu.VMEM((_ROW_LEN,), jnp.int32),
            pltpu.VMEM((3 * _L,), jnp.int32),
            pltpu.VMEM((_ON // 2, _OL), jnp.int32),
            pltpu.VMEM((_ON // 2, _OL), jnp.float32),
            pltpu.VMEM_SHARED((128,), jnp.int32),
            pltpu.VMEM((128,), jnp.int32),
            pltpu.SemaphoreType.DMA,
            pltpu.SemaphoreType.DMA,
        ],
    )(x)
    return otp, lend, mask


# speedup vs baseline: 1.0657x; 1.0041x over previous
import jax
import jax.numpy as jnp
from jax import lax
from jax.experimental import pallas as pl
from jax.experimental.pallas import tpu as pltpu
from jax.experimental.pallas import tpu_sc as plsc

_B, _S = 16, 2048
_ON, _OL = 32, 64
_DELIM, _PAD = 1, 0
_L = 16
_ROW_LEN = _S + _L


def _split_body(x_hbm, otp_hbm, lend_hbm, mask_hbm,
                row_v, beg_v, out_v, msk_v, docs_sh, tmp_v, sem_out, sem_msk):
    cid = lax.axis_index("c")
    sid = lax.axis_index("s")
    row = cid * 8 + sid // 2
    half = sid % 2

    if True:
        lane = lax.broadcasted_iota(jnp.int32, (_L,), 0)
        pieces = [pltpu.async_copy(x_hbm.at[row], row_v.at[pl.ds(0, _S)],
                                   sem_out)]
        row_v[pl.ds(_S, _L)] = jnp.where(lane == 0, _DELIM, _PAD)
        beg_v[pl.ds(0, _L)] = jnp.where(lane == 0, 0, _S + 1)
        beg_v[pl.ds(_L, _L)] = jnp.full((_L,), _S + 1, jnp.int32)
        beg_v[pl.ds(2 * _L, _L)] = jnp.full((_L,), _S + 1, jnp.int32)
        for c in pieces:
            c.wait()

        def scan_cond(c):
            i, found = c
            return (i < _S // _L) & (found < _ON - 1)

        def scan_body(c):
            i, found = c
            v = row_v[pl.ds(i * _L, _L)]
            m = v == _DELIM
            pc = plsc.all_reduce_population_count(m)[0]

            @pl.when(pc != 0)
            def _():
                s = plsc.cumsum(jnp.where(m, 1, 0))
                rank = found + s
                wm = m & (rank <= _ON - 1)
                plsc.store_scatter(beg_v, [jnp.minimum(rank, _ON)],
                                   i * _L + lane + 1, mask=wm)

            return i + 1, found + pc

        lax.while_loop(scan_cond, scan_body, (jnp.int32(0), jnp.int32(0)))

        kbase = half * (_ON // 2)

        def chunk_body(kk, doc):
            k = kbase + kk
            ksplat = jnp.zeros((_L,), jnp.int32) + k
            off = plsc.load_gather(beg_v, [ksplat])
            sz = plsc.load_gather(beg_v, [ksplat + 1]) - off
            ok = sz != 1

            def tok_body(q, ln):
                j = lane + q * _L
                idx = jnp.minimum(off + j, _S)
                g = plsc.load_gather(row_v, [idx])
                val = jnp.where((j < sz) & ok, g, _PAD)
                out_v[kk, pl.ds(q * _L, _L)] = val
                return ln + plsc.all_reduce_population_count(val != 0)

            ln = lax.fori_loop(0, _OL // _L, tok_body,
                               jnp.zeros((_L,), jnp.int32))

            def msk_body(q, _):
                j = lane + q * _L
                msk_v[kk, pl.ds(q * _L, _L)] = jnp.where(
                    j < ln, jnp.float32(1.0), jnp.float32(0.0))
                return 0

            lax.fori_loop(0, _OL // _L, msk_body, 0)
            return doc + jnp.where(ln != 0, 1, 0)

        doc = lax.fori_loop(0, _ON // 2, chunk_body,
                            jnp.zeros((_L,), jnp.int32))
        c1 = pltpu.async_copy(out_v, otp_hbm.at[row, pl.ds(kbase, _ON // 2)],
                              sem_out)
        c2 = pltpu.async_copy(msk_v, mask_hbm.at[row, pl.ds(kbase, _ON // 2)],
                              sem_msk)
        beg_v[pl.ds(0, _L)] = doc
        pltpu.sync_copy(beg_v.at[pl.ds(0, 8)], docs_sh.at[pl.ds(sid * 8, 8)])
        c1.wait()
        c2.wait()

    plsc.subcore_barrier()

    @pl.when(sid == 0)
    def _():
        pltpu.sync_copy(docs_sh, tmp_v)
        idx = lax.broadcasted_iota(jnp.int32, (_L,), 0) * 16
        cnt = (plsc.load_gather(tmp_v, [idx])
               + plsc.load_gather(tmp_v, [idx + 8]))
        tmp_v[pl.ds(0, _L)] = cnt
        pltpu.sync_copy(tmp_v.at[pl.ds(0, 8)], lend_hbm.at[pl.ds(cid * 8, 8)])


@jax.jit
def kernel(x):
    mesh = plsc.VectorSubcoreMesh(core_axis_name="c", subcore_axis_name="s")
    otp, lend, mask = pl.kernel(
        _split_body,
        out_type=[
            jax.ShapeDtypeStruct((_B, _ON, _OL), jnp.int32),
            jax.ShapeDtypeStruct((_B,), jnp.int32),
            jax.ShapeDtypeStruct((_B, _ON, _OL), jnp.float32),
        ],
        mesh=mesh,
        compiler_params=pltpu.CompilerParams(
            needs_layout_passes=False, use_tc_tiling_on_sc=True),
        scratch_types=[
            pltpu.VMEM((_ROW_LEN,), jnp.int32),
            pltpu.VMEM((3 * _L,), jnp.int32),
            pltpu.VMEM((_ON // 2, _OL), jnp.int32),
            pltpu.VMEM((_ON // 2, _OL), jnp.float32),
            pltpu.VMEM_SHARED((128,), jnp.int32),
            pltpu.VMEM((128,), jnp.int32),
            pltpu.SemaphoreType.DMA,
            pltpu.SemaphoreType.DMA,
        ],
    )(x)
    return otp, lend, mask


# speedup vs baseline: 1.0679x; 1.0020x over previous
import jax
import jax.numpy as jnp
from jax import lax
from jax.experimental import pallas as pl
from jax.experimental.pallas import tpu as pltpu
from jax.experimental.pallas import tpu_sc as plsc

_B, _S = 16, 2048
_ON, _OL = 32, 64
_DELIM, _PAD = 1, 0
_L = 16
_ROW_LEN = _S + _L


def _split_body(x_hbm, otp_hbm, lend_hbm, mask_hbm,
                row_v, beg_v, out_v, msk_v, docs_sh, tmp_v, sem_out, sem_msk):
    cid = lax.axis_index("c")
    sid = lax.axis_index("s")
    row = cid * 8 + sid // 2
    half = sid % 2

    if True:
        lane = lax.broadcasted_iota(jnp.int32, (_L,), 0)
        _H = _S // 2
        c_lo = pltpu.async_copy(x_hbm.at[row, pl.ds(0, _H)],
                                row_v.at[pl.ds(0, _H)], sem_out)
        c_hi = pltpu.async_copy(x_hbm.at[row, pl.ds(_H, _H)],
                                row_v.at[pl.ds(_H, _H)], sem_msk)
        row_v[pl.ds(_S, _L)] = jnp.where(lane == 0, _DELIM, _PAD)
        beg_v[pl.ds(0, _L)] = jnp.where(lane == 0, 0, _S + 1)
        beg_v[pl.ds(_L, _L)] = jnp.full((_L,), _S + 1, jnp.int32)
        beg_v[pl.ds(2 * _L, _L)] = jnp.full((_L,), _S + 1, jnp.int32)
        c_lo.wait()

        def scan_cond(c):
            i, found = c
            return (i < _H // _L) & (found < _ON - 1)

        def scan_body(c):
            i, found = c
            v = row_v[pl.ds(i * _L, _L)]
            m = v == _DELIM
            pc = plsc.all_reduce_population_count(m)[0]

            @pl.when(pc != 0)
            def _():
                s = plsc.cumsum(jnp.where(m, 1, 0))
                rank = found + s
                wm = m & (rank <= _ON - 1)
                plsc.store_scatter(beg_v, [jnp.minimum(rank, _ON)],
                                   i * _L + lane + 1, mask=wm)

            return i + 1, found + pc

        st = lax.while_loop(scan_cond, scan_body, (jnp.int32(0), jnp.int32(0)))
        c_hi.wait()

        def scan_cond2(c):
            i, found = c
            return (i < _S // _L) & (found < _ON - 1)

        lax.while_loop(scan_cond2, scan_body, st)

        kbase = half * (_ON // 2)

        def chunk_body(kk, doc):
            k = kbase + kk
            ksplat = jnp.zeros((_L,), jnp.int32) + k
            off = plsc.load_gather(beg_v, [ksplat])
            sz = plsc.load_gather(beg_v, [ksplat + 1]) - off
            ok = sz != 1

            def tok_body(q, ln):
                j = lane + q * _L
                idx = jnp.minimum(off + j, _S)
                g = plsc.load_gather(row_v, [idx])
                val = jnp.where((j < sz) & ok, g, _PAD)
                out_v[kk, pl.ds(q * _L, _L)] = val
                return ln + plsc.all_reduce_population_count(val != 0)

            ln = lax.fori_loop(0, _OL // _L, tok_body,
                               jnp.zeros((_L,), jnp.int32))

            def msk_body(q, _):
                j = lane + q * _L
                msk_v[kk, pl.ds(q * _L, _L)] = jnp.where(
                    j < ln, jnp.float32(1.0), jnp.float32(0.0))
                return 0

            lax.fori_loop(0, _OL // _L, msk_body, 0)
            return doc + jnp.where(ln != 0, 1, 0)

        doc = lax.fori_loop(0, _ON // 2, chunk_body,
                            jnp.zeros((_L,), jnp.int32))
        c1 = pltpu.async_copy(out_v, otp_hbm.at[row, pl.ds(kbase, _ON // 2)],
                              sem_out)
        c2 = pltpu.async_copy(msk_v, mask_hbm.at[row, pl.ds(kbase, _ON // 2)],
                              sem_msk)
        beg_v[pl.ds(0, _L)] = doc
        pltpu.sync_copy(beg_v.at[pl.ds(0, 8)], docs_sh.at[pl.ds(sid * 8, 8)])
        c1.wait()
        c2.wait()

    plsc.subcore_barrier()

    @pl.when(sid == 0)
    def _():
        pltpu.sync_copy(docs_sh, tmp_v)
        idx = lax.broadcasted_iota(jnp.int32, (_L,), 0) * 16
        cnt = (plsc.load_gather(tmp_v, [idx])
               + plsc.load_gather(tmp_v, [idx + 8]))
        tmp_v[pl.ds(0, _L)] = cnt
        pltpu.sync_copy(tmp_v.at[pl.ds(0, 8)], lend_hbm.at[pl.ds(cid * 8, 8)])


@jax.jit
def kernel(x):
    mesh = plsc.VectorSubcoreMesh(core_axis_name="c", subcore_axis_name="s")
    otp, lend, mask = pl.kernel(
        _split_body,
        out_type=[
            jax.ShapeDtypeStruct((_B, _ON, _OL), jnp.int32),
            jax.ShapeDtypeStruct((_B,), jnp.int32),
            jax.ShapeDtypeStruct((_B, _ON, _OL), jnp.float32),
        ],
        mesh=mesh,
        compiler_params=pltpu.CompilerParams(
            needs_layout_passes=False, use_tc_tiling_on_sc=True),
        scratch_types=[
            pltpu.VMEM((_ROW_LEN,), jnp.int32),
            pltpu.VMEM((3 * _L,), jnp.int32),
            pltpu.VMEM((_ON // 2, _OL), jnp.int32),
            pltpu.VMEM((_ON // 2, _OL), jnp.float32),
            pltpu.VMEM_SHARED((128,), jnp.int32),
            pltpu.VMEM((128,), jnp.int32),
            pltpu.SemaphoreType.DMA,
            pltpu.SemaphoreType.DMA,
        ],
    )(x)
    return otp, lend, mask
